# baseline (speedup 1.0000x reference)
import functools

import jax
import jax.numpy as jnp
from jax import lax
from jax.experimental import pallas as pl
from jax.experimental.pallas import tpu as pltpu
from jax.experimental.pallas import tpu_sc as plsc

N_NODES = 10000
N_EDGES = 320000
IN_DIM = 128
HIDDEN_DIM = 128
OUT_DIM = 64
NLAYERS = 2

NC = 2
NS = 16
NW = NC * NS
EDGES_PER_W = N_EDGES // NW
CHUNK = 125
CHUNKS_PER_W = EDGES_PER_W // CHUNK
ROWS_PER_TILE = N_NODES // NS
NBUF = 8
PREF = 4


def _mlp_body(x_ref, w1_ref, w2_ref, o_ref):
    h = jnp.dot(x_ref[...], w1_ref[...], preferred_element_type=jnp.float32)
    h = jnp.maximum(h, 0.0)
    o_ref[...] = jnp.dot(h, w2_ref[...], preferred_element_type=jnp.float32)


_mlp = pl.pallas_call(
    _mlp_body,
    grid=(10,),
    in_specs=[
        pl.BlockSpec((N_NODES // 10, IN_DIM), lambda i: (i, 0)),
        pl.BlockSpec((IN_DIM, HIDDEN_DIM), lambda i: (0, 0)),
        pl.BlockSpec((HIDDEN_DIM, OUT_DIM), lambda i: (0, 0)),
    ],
    out_specs=pl.BlockSpec((N_NODES // 10, OUT_DIM), lambda i: (i, 0)),
    out_shape=jax.ShapeDtypeStruct((N_NODES, OUT_DIM), jnp.float32),
)


def _add_body(a_ref, b_ref, o_ref):
    o_ref[...] = a_ref[...] + b_ref[...]


_add_halves = pl.pallas_call(
    _add_body,
    grid=(10,),
    in_specs=[
        pl.BlockSpec((N_NODES // 10, OUT_DIM), lambda i: (i, 0)),
        pl.BlockSpec((N_NODES // 10, OUT_DIM), lambda i: (i + 10, 0)),
    ],
    out_specs=pl.BlockSpec((N_NODES // 10, OUT_DIM), lambda i: (i, 0)),
    out_shape=jax.ShapeDtypeStruct((N_NODES, OUT_DIM), jnp.float32),
)


_sc_mesh = plsc.VectorSubcoreMesh(
    core_axis_name="c", subcore_axis_name="s", num_cores=NC, num_subcores=NS
)


@functools.partial(
    pl.kernel,
    out_type=jax.ShapeDtypeStruct((NC * N_NODES, OUT_DIM), jnp.float32),
    mesh=_sc_mesh,
    compiler_params=pltpu.CompilerParams(use_tc_tiling_on_sc=False),
    scratch_types=[
        pltpu.VMEM((CHUNKS_PER_W, CHUNK), jnp.int32),
        pltpu.VMEM((CHUNKS_PER_W, CHUNK), jnp.int32),
        pltpu.VMEM((NBUF, CHUNK, OUT_DIM), jnp.float32),
        pltpu.VMEM_SHARED((N_NODES, OUT_DIM), jnp.float32),
        [pltpu.SemaphoreType.DMA] * NBUF,
        [pltpu.SemaphoreType.DMA] * NBUF,
    ],
)
def _propagate(h_hbm, src_hbm, dst_hbm, out_hbm, src_v, dst_v, rows_v,
               acc_sh, gsems, ssems):
    cid = lax.axis_index("c")
    sid = lax.axis_index("s")
    wid = cid * NS + sid

    zeros16 = jnp.zeros((16,), jnp.float32)

    def _zero_row(r, _):
        for b in range(NBUF):
            for l in range(OUT_DIM // 16):
                rows_v[b, r, pl.ds(l * 16, 16)] = zeros16
        return 0

    lax.fori_loop(0, CHUNK, _zero_row, 0)
    for k in range(ROWS_PER_TILE // CHUNK):
        pltpu.sync_copy(
            rows_v.at[k % NBUF],
            acc_sh.at[pl.ds(sid * ROWS_PER_TILE + k * CHUNK, CHUNK)],
        )
    plsc.subcore_barrier()

    pltpu.sync_copy(src_hbm.at[pl.ds(wid * CHUNKS_PER_W, CHUNKS_PER_W)], src_v)
    pltpu.sync_copy(dst_hbm.at[pl.ds(wid * CHUNKS_PER_W, CHUNKS_PER_W)], dst_v)

    def _gather(ci, b):
        return pltpu.make_async_copy(h_hbm.at[src_v.at[ci]], rows_v.at[b],
                                     gsems[b])

    def _scatter_start(ci, b):
        pltpu.async_copy(rows_v.at[b], acc_sh.at[dst_v.at[ci]], ssems[b],
                         add=True)

    def _scatter_wait(ci, b):
        pltpu.make_async_copy(rows_v.at[b], acc_sh.at[dst_v.at[ci]],
                              ssems[b]).wait()

    for b in range(PREF):
        _gather(b, b).start()

    def _step8(k, _):
        i0 = k * NBUF
        for b in range(NBUF):
            ci = i0 + b
            _gather(ci, b).wait()
            _scatter_start(ci, b)
            bn = (b + PREF) % NBUF

            @pl.when(ci + PREF < CHUNKS_PER_W)
            def _():
                @pl.when(ci >= PREF)
                def _():
                    _scatter_wait(ci - PREF, bn)

                _gather(ci + PREF, bn).start()

        return 0

    lax.fori_loop(0, CHUNKS_PER_W // NBUF, _step8, 0)

    for b in range(NBUF):
        _scatter_wait(CHUNKS_PER_W - NBUF + b, b)
    plsc.subcore_barrier()

    pltpu.sync_copy(
        acc_sh.at[pl.ds(sid * ROWS_PER_TILE, ROWS_PER_TILE)],
        out_hbm.at[pl.ds(cid * N_NODES + sid * ROWS_PER_TILE, ROWS_PER_TILE)],
    )


@jax.jit
def kernel(features, edge_index, W1, W2):
    h = _mlp(features, W1, W2)
    edges = edge_index.astype(jnp.int32).reshape(2, N_EDGES // CHUNK, CHUNK)
    src = edges[0]
    dst = edges[1]
    for _ in range(NLAYERS):
        partials = _propagate(h, src, dst)
        h = _add_halves(partials, partials)
    return h


# speedup vs baseline: 1.1575x; 1.1575x over previous
import functools

import jax
import jax.numpy as jnp
from jax import lax
from jax.experimental import pallas as pl
from jax.experimental.pallas import tpu as pltpu
from jax.experimental.pallas import tpu_sc as plsc

N_NODES = 10000
N_EDGES = 320000
IN_DIM = 128
HIDDEN_DIM = 128
OUT_DIM = 64
NLAYERS = 2

NC = 2
NS = 16
NW = NC * NS
EDGES_PER_W = N_EDGES // NW
CHUNK = 128
NFULL = EDGES_PER_W // CHUNK
TAIL = EDGES_PER_W - NFULL * CHUNK
ROWS_PER_TILE = N_NODES // NS
NBUF = 4
PREF = 3


def _mlp_body(x_ref, w1_ref, w2_ref, o_ref):
    h = jnp.dot(x_ref[...], w1_ref[...], preferred_element_type=jnp.float32)
    h = jnp.maximum(h, 0.0)
    o_ref[...] = jnp.dot(h, w2_ref[...], preferred_element_type=jnp.float32)


_mlp = pl.pallas_call(
    _mlp_body,
    grid=(10,),
    in_specs=[
        pl.BlockSpec((N_NODES // 10, IN_DIM), lambda i: (i, 0)),
        pl.BlockSpec((IN_DIM, HIDDEN_DIM), lambda i: (0, 0)),
        pl.BlockSpec((HIDDEN_DIM, OUT_DIM), lambda i: (0, 0)),
    ],
    out_specs=pl.BlockSpec((N_NODES // 10, OUT_DIM), lambda i: (i, 0)),
    out_shape=jax.ShapeDtypeStruct((N_NODES, OUT_DIM), jnp.float32),
)


def _add_body(a_ref, b_ref, o_ref):
    o_ref[...] = a_ref[...] + b_ref[...]


_add_halves = pl.pallas_call(
    _add_body,
    grid=(10,),
    in_specs=[
        pl.BlockSpec((N_NODES // 10, OUT_DIM), lambda i: (i, 0)),
        pl.BlockSpec((N_NODES // 10, OUT_DIM), lambda i: (i + 10, 0)),
    ],
    out_specs=pl.BlockSpec((N_NODES // 10, OUT_DIM), lambda i: (i, 0)),
    out_shape=jax.ShapeDtypeStruct((N_NODES, OUT_DIM), jnp.float32),
)


_sc_mesh = plsc.VectorSubcoreMesh(
    core_axis_name="c", subcore_axis_name="s", num_cores=NC, num_subcores=NS
)


@functools.partial(
    pl.kernel,
    out_type=jax.ShapeDtypeStruct((NC * N_NODES, OUT_DIM), jnp.float32),
    mesh=_sc_mesh,
    compiler_params=pltpu.CompilerParams(use_tc_tiling_on_sc=False),
    scratch_types=[
        pltpu.VMEM((EDGES_PER_W,), jnp.int32),
        pltpu.VMEM((EDGES_PER_W,), jnp.int32),
        pltpu.VMEM((NBUF + 1, CHUNK, OUT_DIM), jnp.float32),
        pltpu.VMEM_SHARED((N_NODES, OUT_DIM), jnp.float32),
        [pltpu.SemaphoreType.DMA] * (NBUF + 1),
    ],
)
def _propagate(h_hbm, edge_hbm, out_hbm, src_v, dst_v, rows_v, acc_sh, gsems):
    cid = lax.axis_index("c")
    sid = lax.axis_index("s")
    wid = cid * NS + sid

    zeros16 = jnp.zeros((16,), jnp.float32)

    def _zero_row(r, _):
        for b in range(NBUF + 1):
            for l in range(OUT_DIM // 16):
                rows_v[b, r, pl.ds(l * 16, 16)] = zeros16
        return 0

    lax.fori_loop(0, CHUNK, _zero_row, 0)
    for k in range(5):
        pltpu.sync_copy(
            rows_v.at[k % NBUF, pl.ds(0, 125)],
            acc_sh.at[pl.ds(sid * ROWS_PER_TILE + k * 125, 125)],
        )
    plsc.subcore_barrier()

    pltpu.sync_copy(edge_hbm.at[0, pl.ds(wid * EDGES_PER_W, EDGES_PER_W)], src_v)
    pltpu.sync_copy(edge_hbm.at[1, pl.ds(wid * EDGES_PER_W, EDGES_PER_W)], dst_v)

    def _gather(ci, b):
        return pltpu.make_async_copy(
            h_hbm.at[src_v.at[pl.ds(ci * CHUNK, CHUNK)]], rows_v.at[b], gsems[b])

    tail_gather = pltpu.make_async_copy(
        h_hbm.at[src_v.at[pl.ds(NFULL * CHUNK, TAIL)]],
        rows_v.at[NBUF, pl.ds(0, TAIL)], gsems[NBUF])
    tail_gather.start()
    for b in range(PREF):
        _gather(b, b).start()

    def _chunk_body(ci, b):
        _gather(ci, b).wait()
        pltpu.sync_copy(rows_v.at[b],
                        acc_sh.at[dst_v.at[pl.ds(ci * CHUNK, CHUNK)]],
                        add=True)

        @pl.when(ci + PREF < NFULL)
        def _():
            _gather(ci + PREF, (b + PREF) % NBUF).start()

    def _step(k, _):
        i0 = k * NBUF
        for b in range(NBUF):
            _chunk_body(i0 + b, b)
        return 0

    lax.fori_loop(0, NFULL // NBUF, _step, 0)
    for ci in range((NFULL // NBUF) * NBUF, NFULL):
        _chunk_body(ci, ci % NBUF)

    tail_gather.wait()
    pltpu.sync_copy(rows_v.at[NBUF, pl.ds(0, TAIL)],
                    acc_sh.at[dst_v.at[pl.ds(NFULL * CHUNK, TAIL)]],
                    add=True)
    plsc.subcore_barrier()

    pltpu.sync_copy(
        acc_sh.at[pl.ds(sid * ROWS_PER_TILE, ROWS_PER_TILE)],
        out_hbm.at[pl.ds(cid * N_NODES + sid * ROWS_PER_TILE, ROWS_PER_TILE)],
    )


@jax.jit
def kernel(features, edge_index, W1, W2):
    h = _mlp(features, W1, W2)
    edges = edge_index.astype(jnp.int32)
    for _ in range(NLAYERS):
        partials = _propagate(h, edges)
        h = _add_halves(partials, partials)
    return h
